# Initial kernel scaffold; baseline (speedup 1.0000x reference)
import functools

import jax
import jax.numpy as jnp
from jax import lax
from jax.experimental import pallas as pl
from jax.experimental.pallas import tpu as pltpu
from jax.experimental.pallas import tpu_sc as plsc

VOCAB = 100000
MAX_LEN = 200
EMBED = 64
B = 4096
L = 200

NC = 2
NS = 16
NW = NC * NS

ROWS = B * L
ROWS_PER_W = ROWS // NW
CHUNK = 100
CHUNKS_PER_W = ROWS_PER_W // CHUNK
NBUF = 4
NGROUPS = CHUNKS_PER_W // NBUF


def _body(x_hbm, word_hbm, pos_hbm, out_hbm,
          idx_v, pos_v, g0, g1, g2, g3, o0, o1,
          gs0, gs1, gs2, gs3, ss0, ss1):
  gbuf = (g0, g1, g2, g3)
  obuf = (o0, o1)
  gsem = (gs0, gs1, gs2, gs3)
  ssem = (ss0, ss1)

  cid = lax.axis_index("c")
  sid = lax.axis_index("s")
  wid = sid * NC + cid
  idx_base = wid * CHUNKS_PER_W
  row_base = wid * ROWS_PER_W

  pltpu.sync_copy(x_hbm.at[pl.ds(idx_base, CHUNKS_PER_W)], idx_v)
  pltpu.sync_copy(pos_hbm, pos_v)

  def gather_start(c, b):
    pltpu.async_copy(word_hbm.at[idx_v.at[c]], gbuf[b], gsem[b])

  def gather_wait(b):
    pltpu.make_async_copy(word_hbm.at[idx_v.at[0]], gbuf[b], gsem[b]).wait()

  def store_start(c, ob):
    pltpu.async_copy(obuf[ob], out_hbm.at[pl.ds(row_base + c * CHUNK, CHUNK)],
                     ssem[ob])

  def store_wait(ob):
    pltpu.make_async_copy(obuf[ob], out_hbm.at[pl.ds(row_base, CHUNK)],
                          ssem[ob]).wait()

  for b in range(NBUF):
    gather_start(b, b)

  def group(g, carry):
    for b in range(NBUF):
      c = g * NBUF + b
      gather_wait(b)
      ob = b % 2
      if b >= 2:
        store_wait(ob)
      else:
        @pl.when(g > 0)
        def _():
          store_wait(ob)
      phase = (b % 2) * CHUNK

      def add_row(r, acc):
        for k in range(EMBED // 16):
          sl = pl.ds(k * 16, 16)
          obuf[ob][r, sl] = gbuf[b][r, sl] + pos_v[phase + r, sl]
        return acc

      lax.fori_loop(0, CHUNK, add_row, 0, unroll=2)
      store_start(c, ob)

      @pl.when(g < NGROUPS - 1)
      def _():
        gather_start(c + NBUF, b)
    return carry

  lax.fori_loop(0, NGROUPS, group, 0)
  store_wait(0)
  store_wait(1)


@jax.jit
def _emb(x2, word_table, pos_table):
  mesh = plsc.VectorSubcoreMesh(core_axis_name="c", subcore_axis_name="s")
  return pl.kernel(
      _body,
      out_type=jax.ShapeDtypeStruct((ROWS, EMBED), jnp.float32),
      mesh=mesh,
      scratch_types=[
          pltpu.VMEM((CHUNKS_PER_W, CHUNK), jnp.int32),
          pltpu.VMEM((MAX_LEN, EMBED), jnp.float32),
          pltpu.VMEM((CHUNK, EMBED), jnp.float32),
          pltpu.VMEM((CHUNK, EMBED), jnp.float32),
          pltpu.VMEM((CHUNK, EMBED), jnp.float32),
          pltpu.VMEM((CHUNK, EMBED), jnp.float32),
          pltpu.VMEM((CHUNK, EMBED), jnp.float32),
          pltpu.VMEM((CHUNK, EMBED), jnp.float32),
          pltpu.SemaphoreType.DMA,
          pltpu.SemaphoreType.DMA,
          pltpu.SemaphoreType.DMA,
          pltpu.SemaphoreType.DMA,
          pltpu.SemaphoreType.DMA,
          pltpu.SemaphoreType.DMA,
      ],
  )(x2, word_table, pos_table)


def kernel(x, word_table, pos_table):
  x2 = x.reshape(ROWS // CHUNK, CHUNK).astype(jnp.int32)
  out = _emb(x2, word_table, pos_table)
  return out.reshape(B, L, EMBED)


# speedup vs baseline: 5.2555x; 5.2555x over previous
import jax
import jax.numpy as jnp
from jax import lax
from jax.experimental import pallas as pl
from jax.experimental.pallas import tpu as pltpu
from jax.experimental.pallas import tpu_sc as plsc

VOCAB = 100000
MAX_LEN = 200
EMBED = 64
B = 4096
L = 200

NC = 2
NS = 16
NW = NC * NS

ROWS = B * L
ROWS_PER_W = ROWS // NW
CHUNK = 40
NBUF = L // CHUNK
CHUNKS_PER_W = ROWS_PER_W // CHUNK
NGROUPS = CHUNKS_PER_W // NBUF


def _body(x_hbm, word_hbm, pos_hbm, out_hbm, idx_v, pos_v, *rest):
  gbuf = rest[0:NBUF]
  obuf = rest[NBUF:2 * NBUF]
  gsem = rest[2 * NBUF:3 * NBUF]
  ssem = rest[3 * NBUF:4 * NBUF]

  cid = lax.axis_index("c")
  sid = lax.axis_index("s")
  wid = sid * NC + cid
  idx_base = wid * CHUNKS_PER_W
  row_base = wid * ROWS_PER_W

  pltpu.sync_copy(x_hbm.at[pl.ds(idx_base, CHUNKS_PER_W)], idx_v)
  pltpu.sync_copy(pos_hbm, pos_v)

  def gather_start(c, b):
    pltpu.async_copy(word_hbm.at[idx_v.at[c]], gbuf[b], gsem[b])

  def gather_wait(b):
    pltpu.make_async_copy(word_hbm.at[idx_v.at[0]], gbuf[b], gsem[b]).wait()

  def store_start(c, b):
    pltpu.async_copy(obuf[b], out_hbm.at[pl.ds(row_base + c * CHUNK, CHUNK)],
                     ssem[b])

  def store_wait(b):
    pltpu.make_async_copy(obuf[b], out_hbm.at[pl.ds(row_base, CHUNK)],
                          ssem[b]).wait()

  for b in range(NBUF):
    gather_start(b, b)

  def group(g, carry):
    for b in range(NBUF):
      c = g * NBUF + b
      gather_wait(b)

      @pl.when(g > 0)
      def _():
        store_wait(b)

      phase = b * CHUNK

      def add_row(r, acc):
        for k in range(EMBED // 16):
          sl = pl.ds(k * 16, 16)
          obuf[b][r, sl] = gbuf[b][r, sl] + pos_v[phase + r, sl]
        return acc

      lax.fori_loop(0, CHUNK, add_row, 0, unroll=2)
      store_start(c, b)

      @pl.when(g < NGROUPS - 1)
      def _():
        gather_start(c + NBUF, b)
    return carry

  lax.fori_loop(0, NGROUPS, group, 0)
  for b in range(NBUF):
    store_wait(b)


@jax.jit
def _emb(x2, word_table, pos_table):
  mesh = plsc.VectorSubcoreMesh(core_axis_name="c", subcore_axis_name="s")
  return pl.kernel(
      _body,
      out_type=jax.ShapeDtypeStruct((ROWS, EMBED), jnp.float32),
      mesh=mesh,
      compiler_params=pltpu.CompilerParams(use_tc_tiling_on_sc=False),
      scratch_types=(
          [pltpu.VMEM((CHUNKS_PER_W, CHUNK), jnp.int32),
           pltpu.VMEM((MAX_LEN, EMBED), jnp.float32)]
          + [pltpu.VMEM((CHUNK, EMBED), jnp.float32)] * (2 * NBUF)
          + [pltpu.SemaphoreType.DMA] * (2 * NBUF)
      ),
  )(x2, word_table, pos_table)


def kernel(x, word_table, pos_table):
  x2 = x.reshape(ROWS // CHUNK, CHUNK).astype(jnp.int32)
  out = _emb(x2, word_table, pos_table)
  return out.reshape(B, L, EMBED)


# speedup vs baseline: 5.2570x; 1.0003x over previous
import jax
import jax.numpy as jnp
from jax import lax
from jax.experimental import pallas as pl
from jax.experimental.pallas import tpu as pltpu
from jax.experimental.pallas import tpu_sc as plsc

VOCAB = 100000
MAX_LEN = 200
EMBED = 64
B = 4096
L = 200

NC = 2
NS = 16
NW = NC * NS

ROWS = B * L
ROWS_PER_W = ROWS // NW
CHUNK = 128
NBUF = 4
CHUNKS_PER_W = ROWS_PER_W // CHUNK
NGROUPS = CHUNKS_PER_W // NBUF


def _body(x_hbm, word_hbm, pos_hbm, out_hbm, idx_v, pos_v, *rest):
  gbuf = rest[0:NBUF]
  obuf = rest[NBUF:2 * NBUF]
  gsem = rest[2 * NBUF:3 * NBUF]
  ssem = rest[3 * NBUF:4 * NBUF]

  cid = lax.axis_index("c")
  sid = lax.axis_index("s")
  wid = sid * NC + cid
  idx_base = wid * CHUNKS_PER_W
  row_base = wid * ROWS_PER_W

  pltpu.sync_copy(x_hbm.at[pl.ds(idx_base, CHUNKS_PER_W)], idx_v)
  pltpu.sync_copy(pos_hbm, pos_v.at[pl.ds(0, MAX_LEN)])
  pltpu.sync_copy(pos_hbm, pos_v.at[pl.ds(MAX_LEN, MAX_LEN)])

  def gather_start(c, b):
    pltpu.async_copy(word_hbm.at[idx_v.at[c]], gbuf[b], gsem[b])

  def gather_wait(b):
    pltpu.make_async_copy(word_hbm.at[idx_v.at[0]], gbuf[b], gsem[b]).wait()

  def store_start(c, b):
    pltpu.async_copy(obuf[b], out_hbm.at[pl.ds(row_base + c * CHUNK, CHUNK)],
                     ssem[b])

  def store_wait(b):
    pltpu.make_async_copy(obuf[b], out_hbm.at[pl.ds(row_base, CHUNK)],
                          ssem[b]).wait()

  for b in range(NBUF):
    gather_start(b, b)

  def group(g, carry):
    for b in range(NBUF):
      c = g * NBUF + b
      gather_wait(b)

      @pl.when(g > 0)
      def _():
        store_wait(b)

      phase = lax.rem(c * CHUNK, MAX_LEN)

      def add_row(r, acc):
        p = phase + r
        for k in range(EMBED // 16):
          sl = pl.ds(k * 16, 16)
          obuf[b][r, sl] = gbuf[b][r, sl] + pos_v[p, sl]
        return acc

      lax.fori_loop(0, CHUNK, add_row, 0, unroll=2)
      store_start(c, b)

      @pl.when(g < NGROUPS - 1)
      def _():
        gather_start(c + NBUF, b)
    return carry

  lax.fori_loop(0, NGROUPS, group, 0)
  for b in range(NBUF):
    store_wait(b)


@jax.jit
def _emb(x2, word_table, pos_table):
  mesh = plsc.VectorSubcoreMesh(core_axis_name="c", subcore_axis_name="s")
  return pl.kernel(
      _body,
      out_type=jax.ShapeDtypeStruct((ROWS, EMBED), jnp.float32),
      mesh=mesh,
      compiler_params=pltpu.CompilerParams(use_tc_tiling_on_sc=False),
      scratch_types=(
          [pltpu.VMEM((CHUNKS_PER_W, CHUNK), jnp.int32),
           pltpu.VMEM((2 * MAX_LEN, EMBED), jnp.float32)]
          + [pltpu.VMEM((CHUNK, EMBED), jnp.float32)] * (2 * NBUF)
          + [pltpu.SemaphoreType.DMA] * (2 * NBUF)
      ),
  )(x2, word_table, pos_table)


def kernel(x, word_table, pos_table):
  x2 = x.reshape(ROWS // CHUNK, CHUNK).astype(jnp.int32)
  out = _emb(x2, word_table, pos_table)
  return out.reshape(B, L, EMBED)


# speedup vs baseline: 5.2732x; 1.0031x over previous
import jax
import jax.numpy as jnp
from jax import lax
from jax.experimental import pallas as pl
from jax.experimental.pallas import tpu as pltpu
from jax.experimental.pallas import tpu_sc as plsc

VOCAB = 100000
MAX_LEN = 200
EMBED = 64
B = 4096
L = 200

NC = 2
NS = 16
NW = NC * NS

SEQ_PER_W = B // NW
CHUNK_A = 128
CHUNK_B = L - CHUNK_A
SLOTS = ((0, 0, CHUNK_A), (0, CHUNK_A, CHUNK_B),
         (1, 0, CHUNK_A), (1, CHUNK_A, CHUNK_B))
NSLOT = len(SLOTS)
NGROUPS = SEQ_PER_W // 2


def _body(x_hbm, word_hbm, pos_hbm, out_hbm, idx_v, pos_v, *rest):
  gbuf = rest[0:NSLOT]
  obuf = rest[NSLOT:2 * NSLOT]
  gsem = rest[2 * NSLOT:3 * NSLOT]
  ssem = rest[3 * NSLOT:4 * NSLOT]

  cid = lax.axis_index("c")
  sid = lax.axis_index("s")
  wid = sid * NC + cid
  seq_base = wid * SEQ_PER_W

  pltpu.sync_copy(x_hbm.at[pl.ds(seq_base, SEQ_PER_W)], idx_v)
  pltpu.sync_copy(pos_hbm, pos_v)

  def gather_start(g, s):
    dseq, off, ln = SLOTS[s]
    pltpu.async_copy(word_hbm.at[idx_v.at[2 * g + dseq, pl.ds(off, ln)]],
                     gbuf[s], gsem[s])

  def gather_wait(s):
    pltpu.make_async_copy(word_hbm.at[idx_v.at[0, pl.ds(0, SLOTS[s][2])]],
                          gbuf[s], gsem[s]).wait()

  def store_start(g, s):
    dseq, off, ln = SLOTS[s]
    pltpu.async_copy(obuf[s],
                     out_hbm.at[seq_base + 2 * g + dseq, pl.ds(off, ln)],
                     ssem[s])

  def store_wait(s):
    _, off, ln = SLOTS[s]
    pltpu.make_async_copy(obuf[s], out_hbm.at[0, pl.ds(off, ln)],
                          ssem[s]).wait()

  for s in range(NSLOT):
    gather_start(0, s)

  def group(g, carry):
    for s in range(NSLOT):
      _, off, ln = SLOTS[s]
      gather_wait(s)

      @pl.when(g > 0)
      def _():
        store_wait(s)

      def add_row(r, acc):
        for k in range(EMBED // 16):
          sl = pl.ds(k * 16, 16)
          obuf[s][r, sl] = gbuf[s][r, sl] + pos_v[off + r, sl]
        return acc

      lax.fori_loop(0, ln, add_row, 0, unroll=2)
      store_start(g, s)

      @pl.when(g < NGROUPS - 1)
      def _():
        gather_start(g + 1, s)
    return carry

  lax.fori_loop(0, NGROUPS, group, 0)
  for s in range(NSLOT):
    store_wait(s)


@jax.jit
def _emb(x, word_table, pos_table):
  mesh = plsc.VectorSubcoreMesh(core_axis_name="c", subcore_axis_name="s")
  return pl.kernel(
      _body,
      out_type=jax.ShapeDtypeStruct((B, L, EMBED), jnp.float32),
      mesh=mesh,
      compiler_params=pltpu.CompilerParams(use_tc_tiling_on_sc=False),
      scratch_types=(
          [pltpu.VMEM((SEQ_PER_W, L), jnp.int32),
           pltpu.VMEM((MAX_LEN, EMBED), jnp.float32)]
          + [pltpu.VMEM((ln, EMBED), jnp.float32) for _, _, ln in SLOTS] * 2
          + [pltpu.SemaphoreType.DMA] * (2 * NSLOT)
      ),
  )(x, word_table, pos_table)


def kernel(x, word_table, pos_table):
  return _emb(x.astype(jnp.int32), word_table, pos_table)
